# Initial kernel scaffold; baseline (speedup 1.0000x reference)
import jax
import jax.numpy as jnp
from jax.experimental import pallas as pl


def kernel(coords, identity_probs):
    raise NotImplementedError("write your pallas kernel here")


# speedup vs baseline: 1.1655x; 1.1655x over previous
import jax
import jax.numpy as jnp
from jax.experimental import pallas as pl
from jax.experimental.pallas import tpu as pltpu

_N = 8192
_P = 256
_RADIUS2 = 2500.0
_INV_2SIG2 = 1.0 / (2.0 * 25.0 * 25.0)

_BI = 1024
_BJ = 1024


def _loss_kernel(ci_ref, pi_ref, cj_ref, pj_ref, out_ref,
                 q_acc, wsum_acc, wp2_acc):
    i = pl.program_id(0)
    j = pl.program_id(1)
    nj = pl.num_programs(1)

    @pl.when(j == 0)
    def _init():
        q_acc[...] = jnp.zeros_like(q_acc)
        wsum_acc[...] = jnp.zeros_like(wsum_acc)
        wp2_acc[...] = jnp.zeros_like(wp2_acc)

    ci = ci_ref[...]
    cj = cj_ref[...]
    c2i = jnp.sum(ci * ci, axis=1, keepdims=True)
    c2j = jnp.sum(cj * cj, axis=1)
    dot = jax.lax.dot_general(
        ci, cj, (((1,), (1,)), ((), ())),
        preferred_element_type=jnp.float32)
    d2 = c2i + c2j[None, :] - 2.0 * dot
    d2 = jnp.maximum(d2, 0.0)
    mask = (d2 > 0.0) & (d2 <= _RADIUS2)
    w = jnp.where(mask, jnp.exp(-d2 * _INV_2SIG2), 0.0)

    pj = pj_ref[...]
    p2j = jnp.sum(pj * pj, axis=1, keepdims=True)
    q_acc[...] += jax.lax.dot_general(
        w, pj, (((1,), (0,)), ((), ())),
        preferred_element_type=jnp.float32)
    wsum_acc[...] += jnp.sum(w, axis=1, keepdims=True)
    wp2_acc[...] += jax.lax.dot_general(
        w, p2j, (((1,), (0,)), ((), ())),
        preferred_element_type=jnp.float32)

    @pl.when(j == nj - 1)
    def _finalize():
        pi = pi_ref[...]
        p2i = jnp.sum(pi * pi, axis=1, keepdims=True)
        pq = jnp.sum(pi * q_acc[...], axis=1, keepdims=True)
        wsum = wsum_acc[...]
        num = p2i * wsum + wp2_acc[...] - 2.0 * pq
        denom = jnp.where(wsum > 0.0, wsum, 1.0)
        contrib = jnp.sum(num / denom)

        @pl.when(i == 0)
        def _():
            out_ref[...] = jnp.full((1, 1), contrib, jnp.float32)

        @pl.when(i != 0)
        def _():
            out_ref[...] += jnp.full((1, 1), contrib, jnp.float32)


def kernel(coords, identity_probs):
    ni = _N // _BI
    nj = _N // _BJ
    out = pl.pallas_call(
        _loss_kernel,
        grid=(ni, nj),
        in_specs=[
            pl.BlockSpec((_BI, 2), lambda i, j: (i, 0)),
            pl.BlockSpec((_BI, _P), lambda i, j: (i, 0)),
            pl.BlockSpec((_BJ, 2), lambda i, j: (j, 0)),
            pl.BlockSpec((_BJ, _P), lambda i, j: (j, 0)),
        ],
        out_specs=pl.BlockSpec((1, 1), lambda i, j: (0, 0)),
        out_shape=jax.ShapeDtypeStruct((1, 1), jnp.float32),
        scratch_shapes=[
            pltpu.VMEM((_BI, _P), jnp.float32),
            pltpu.VMEM((_BI, 1), jnp.float32),
            pltpu.VMEM((_BI, 1), jnp.float32),
        ],
    )(coords, identity_probs, coords, identity_probs)
    return out[0, 0] / _N


# speedup vs baseline: 1.6066x; 1.3785x over previous
import jax
import jax.numpy as jnp
from jax.experimental import pallas as pl
from jax.experimental.pallas import tpu as pltpu

_N = 8192
_P = 256
_RADIUS2 = 2500.0
_INV_2SIG2 = 1.0 / (2.0 * 25.0 * 25.0)

_BI = 1024
_BJ = 1024


def _loss_kernel(ci_ref, pi_ref, cj_ref, pj_ref, out_ref,
                 q_acc, w2_acc):
    i = pl.program_id(0)
    j = pl.program_id(1)
    nj = pl.num_programs(1)

    @pl.when(j == 0)
    def _init():
        q_acc[...] = jnp.zeros_like(q_acc)
        w2_acc[...] = jnp.zeros_like(w2_acc)

    ci = ci_ref[...]
    cj = cj_ref[...]
    c2i = jnp.sum(ci * ci, axis=1, keepdims=True)
    c2j = jnp.sum(cj * cj, axis=1)
    dot = jax.lax.dot_general(
        ci, cj, (((1,), (1,)), ((), ())),
        preferred_element_type=jnp.float32)
    d2 = c2i + c2j[None, :] - 2.0 * dot
    mask = (d2 > 0.0) & (d2 <= _RADIUS2)
    w = jnp.where(mask, jnp.exp(-d2 * _INV_2SIG2), 0.0)

    pj = pj_ref[...]
    p2j = jnp.sum(pj * pj, axis=1, keepdims=True)
    q_acc[...] += jax.lax.dot_general(
        w, pj, (((1,), (0,)), ((), ())),
        preferred_element_type=jnp.float32)
    m2 = jnp.concatenate([p2j, jnp.ones_like(p2j)], axis=1)
    w2_acc[...] += jax.lax.dot_general(
        w, m2, (((1,), (0,)), ((), ())),
        preferred_element_type=jnp.float32)

    @pl.when(j == nj - 1)
    def _finalize():
        pi = pi_ref[...]
        p2i = jnp.sum(pi * pi, axis=1, keepdims=True)
        pq = jnp.sum(pi * q_acc[...], axis=1, keepdims=True)
        wsum = w2_acc[:, 1:2]
        num = p2i * wsum + w2_acc[:, 0:1] - 2.0 * pq
        denom = jnp.where(wsum > 0.0, wsum, 1.0)
        contrib = jnp.sum(num / denom)

        @pl.when(i == 0)
        def _():
            out_ref[...] = jnp.full((1, 1), contrib, jnp.float32)

        @pl.when(i != 0)
        def _():
            out_ref[...] += jnp.full((1, 1), contrib, jnp.float32)


def kernel(coords, identity_probs):
    ni = _N // _BI
    nj = _N // _BJ
    out = pl.pallas_call(
        _loss_kernel,
        grid=(ni, nj),
        in_specs=[
            pl.BlockSpec((_BI, 2), lambda i, j: (i, 0)),
            pl.BlockSpec((_BI, _P), lambda i, j: (i, 0)),
            pl.BlockSpec((_BJ, 2), lambda i, j: (j, 0)),
            pl.BlockSpec((_BJ, _P), lambda i, j: (j, 0)),
        ],
        out_specs=pl.BlockSpec((1, 1), lambda i, j: (0, 0)),
        out_shape=jax.ShapeDtypeStruct((1, 1), jnp.float32),
        scratch_shapes=[
            pltpu.VMEM((_BI, _P), jnp.float32),
            pltpu.VMEM((_BI, 2), jnp.float32),
        ],
    )(coords, identity_probs, coords, identity_probs)
    return out[0, 0] / _N
